# baseline (speedup 1.0000x reference)
import functools

import jax
import jax.numpy as jnp
from jax import lax
from jax.experimental import pallas as pl
from jax.experimental.pallas import tpu as pltpu
from jax.experimental.pallas import tpu_sc as plsc

B, S, D = 4, 2048, 128
NC, NS, L = 2, 16, 16
NW = NC * NS
BPW = (B * S) // NW
GCH = 128
NG = BPW // GCH


def _emb_body(tok_hbm, emb_hbm, pos_hbm, out_hbm, idx_v, rows_v, pos_v,
              gsem0, gsem1, psem, ssem):
    wid = lax.axis_index("s") * NC + lax.axis_index("c")
    base = wid * BPW
    pos_start = lax.rem(base, S)

    pltpu.sync_copy(tok_hbm.at[wid], idx_v)

    gsems = (gsem0, gsem1)
    gathers = [
        pltpu.async_copy(
            emb_hbm.at[idx_v.at[j]], rows_v.at[pl.ds(j * GCH, GCH)], gsems[j]
        )
        for j in range(NG)
    ]
    pltpu.async_copy(pos_hbm.at[pl.ds(pos_start, BPW)], pos_v, psem).wait()

    def add_rows(r, _):
        for rr in range(2):
            for j in range(D // L):
                sl = pl.ds(j * L, L)
                plsc.addupdate(rows_v.at[r * 2 + rr, sl], pos_v[r * 2 + rr, sl])
        return 0

    stores = []
    for j in range(NG):
        gathers[j].wait()
        lax.fori_loop(j * (GCH // 2), (j + 1) * (GCH // 2), add_rows, 0)
        stores.append(
            pltpu.async_copy(
                rows_v.at[pl.ds(j * GCH, GCH)],
                out_hbm.at[pl.ds(base + j * GCH, GCH)],
                ssem,
            )
        )
    for st in stores:
        st.wait()


@jax.jit
def _emb_call(tokens_flat, emb_table, pos_table):
    mesh = plsc.VectorSubcoreMesh(core_axis_name="c", subcore_axis_name="s")
    call = functools.partial(
        pl.kernel,
        mesh=mesh,
        out_type=jax.ShapeDtypeStruct((B * S, D), jnp.float32),
        scratch_types=[
            pltpu.VMEM((NG, GCH), jnp.int32),
            pltpu.VMEM((BPW, D), jnp.float32),
            pltpu.VMEM((BPW, D), jnp.float32),
            pltpu.SemaphoreType.DMA,
            pltpu.SemaphoreType.DMA,
            pltpu.SemaphoreType.DMA,
            pltpu.SemaphoreType.DMA,
        ],
    )(_emb_body)
    return call(tokens_flat, emb_table, pos_table)


def kernel(tokens, emb_table, pos_table):
    tokens_flat = tokens.astype(jnp.int32).reshape(NW, NG, GCH)
    out = _emb_call(tokens_flat, emb_table, pos_table)
    return out.reshape(B, S, D)


# speedup vs baseline: 1.0296x; 1.0296x over previous
import functools

import jax
import jax.numpy as jnp
from jax import lax
from jax.experimental import pallas as pl
from jax.experimental.pallas import tpu as pltpu
from jax.experimental.pallas import tpu_sc as plsc

B, S, D = 4, 2048, 128
NC, NS, L = 2, 16, 16
NW = NC * NS
BPW = (B * S) // NW
GCH = 128
NG = BPW // GCH


def _emb_body(tok_hbm, emb_hbm, pos_hbm, out_hbm, idx_v, rows_v, pos_v,
              gsem0, gsem1, psem, ssem):
    wid = lax.axis_index("s") * NC + lax.axis_index("c")
    base = wid * BPW
    pos_start = lax.rem(base, S)

    pltpu.sync_copy(tok_hbm.at[wid], idx_v)

    gsems = (gsem0, gsem1)
    gathers = []
    for j in range(NG):
        pltpu.sync_copy(
            pos_hbm.at[pl.ds(pos_start + j * GCH, GCH)],
            rows_v.at[pl.ds(j * GCH, GCH)],
        )
        gathers.append(
            pltpu.async_copy(
                emb_hbm.at[idx_v.at[j]],
                rows_v.at[pl.ds(j * GCH, GCH)],
                gsems[j],
                add=True,
            )
        )
    stores = []
    for j in range(NG):
        gathers[j].wait()
        stores.append(
            pltpu.async_copy(
                rows_v.at[pl.ds(j * GCH, GCH)],
                out_hbm.at[pl.ds(base + j * GCH, GCH)],
                ssem,
            )
        )
    for st in stores:
        st.wait()


@jax.jit
def _emb_call(tokens_flat, emb_table, pos_table):
    mesh = plsc.VectorSubcoreMesh(core_axis_name="c", subcore_axis_name="s")
    call = functools.partial(
        pl.kernel,
        mesh=mesh,
        out_type=jax.ShapeDtypeStruct((B * S, D), jnp.float32),
        scratch_types=[
            pltpu.VMEM((NG, GCH), jnp.int32),
            pltpu.VMEM((BPW, D), jnp.float32),
            pltpu.VMEM((BPW, D), jnp.float32),
            pltpu.SemaphoreType.DMA,
            pltpu.SemaphoreType.DMA,
            pltpu.SemaphoreType.DMA,
            pltpu.SemaphoreType.DMA,
        ],
    )(_emb_body)
    return call(tokens_flat, emb_table, pos_table)


def kernel(tokens, emb_table, pos_table):
    tokens_flat = tokens.astype(jnp.int32).reshape(NW, NG, GCH)
    out = _emb_call(tokens_flat, emb_table, pos_table)
    return out.reshape(B, S, D)


# speedup vs baseline: 1.0671x; 1.0364x over previous
import functools

import jax
import jax.numpy as jnp
from jax import lax
from jax.experimental import pallas as pl
from jax.experimental.pallas import tpu as pltpu
from jax.experimental.pallas import tpu_sc as plsc

B, S, D = 4, 2048, 128
NC, NS, L = 2, 16, 16
NW = NC * NS
BPW = (B * S) // NW
NCH = 4
CH = BPW // NCH


def _emb_body(tok_hbm, emb_hbm, pos_hbm, out_hbm, idx_v, rows_v,
              p0, p1, p2, p3, g0, g1, g2, g3, ssem):
    wid = lax.axis_index("s") * NC + lax.axis_index("c")
    base = wid * BPW
    pos_start = lax.rem(base, S)
    psems = (p0, p1, p2, p3)
    gsems = (g0, g1, g2, g3)

    pcopies = [
        pltpu.async_copy(
            pos_hbm.at[pl.ds(pos_start + j * CH, CH)],
            rows_v.at[pl.ds(j * CH, CH)],
            psems[j],
        )
        for j in range(NCH)
    ]
    pltpu.sync_copy(tok_hbm.at[wid], idx_v)

    gadds = []
    for j in range(NCH):
        pcopies[j].wait()
        gadds.append(
            pltpu.async_copy(
                emb_hbm.at[idx_v.at[j]],
                rows_v.at[pl.ds(j * CH, CH)],
                gsems[j],
                add=True,
            )
        )
    stores = []
    for j in range(NCH):
        gadds[j].wait()
        stores.append(
            pltpu.async_copy(
                rows_v.at[pl.ds(j * CH, CH)],
                out_hbm.at[pl.ds(base + j * CH, CH)],
                ssem,
            )
        )
    for st in stores:
        st.wait()


@jax.jit
def _emb_call(tokens_flat, emb_table, pos_table):
    mesh = plsc.VectorSubcoreMesh(core_axis_name="c", subcore_axis_name="s")
    call = functools.partial(
        pl.kernel,
        mesh=mesh,
        out_type=jax.ShapeDtypeStruct((B * S, D), jnp.float32),
        scratch_types=[
            pltpu.VMEM((NCH, CH), jnp.int32),
            pltpu.VMEM((BPW, D), jnp.float32),
        ] + [pltpu.SemaphoreType.DMA] * 9,
    )(_emb_body)
    return call(tokens_flat, emb_table, pos_table)


def kernel(tokens, emb_table, pos_table):
    tokens_flat = tokens.astype(jnp.int32).reshape(NW, NCH, CH)
    out = _emb_call(tokens_flat, emb_table, pos_table)
    return out.reshape(B, S, D)
